# baseline (speedup 1.0000x reference)
import functools

import jax
import jax.numpy as jnp
from jax import lax
from jax.experimental import pallas as pl
from jax.experimental.pallas import tpu as pltpu
from jax.experimental.pallas import tpu_sc as plsc

EMBED = 128
ROWS = 4096
WIDTH = 13 * EMBED

_PLANS = []
for _start in (0, 1):
    _plan = []
    for _j, _i in enumerate(range(_start, 26, 2)):
        _plan.append((0 if _i < 13 else 1, (_i % 13) * EMBED, _j * EMBED))
    _PLANS.append(tuple(_plan))
_PLANS = tuple(_PLANS)


def _make_kernel():
    info = plsc.get_sparse_core_info()
    nc, ns = info.num_cores, info.num_subcores
    nw = nc * ns
    rpw = ROWS // nw

    mesh = plsc.VectorSubcoreMesh(core_axis_name="c", subcore_axis_name="s")

    @functools.partial(
        pl.kernel,
        mesh=mesh,
        out_type=[
            jax.ShapeDtypeStruct((ROWS, WIDTH), jnp.float32),
            jax.ShapeDtypeStruct((ROWS, WIDTH), jnp.float32),
        ],
    )
    def regroup(kt0, kt1, out_even, out_odd):
        wid = lax.axis_index("s") * nc + lax.axis_index("c")
        r0 = wid * rpw
        srcs = (kt0, kt1)
        for out, plan in ((out_even, _PLANS[0]), (out_odd, _PLANS[1])):
            for src, sc, dc in plan:
                pltpu.sync_copy(
                    srcs[src].at[pl.ds(r0, rpw), pl.ds(sc, EMBED)],
                    out.at[pl.ds(r0, rpw), pl.ds(dc, EMBED)],
                )

    return regroup


_REGROUP = None


def kernel(kt0_values, kt1_values):
    global _REGROUP
    if _REGROUP is None:
        _REGROUP = _make_kernel()
    return tuple(_REGROUP(kt0_values, kt1_values))


# speedup vs baseline: 26.7068x; 26.7068x over previous
import functools

import jax
import jax.numpy as jnp
from jax import lax
from jax.experimental import pallas as pl
from jax.experimental.pallas import tpu as pltpu
from jax.experimental.pallas import tpu_sc as plsc

EMBED = 128
ROWS = 4096
WIDTH = 13 * EMBED
CHUNK = 32

_PLANS = []
for _start in (0, 1):
    _plan = []
    for _j, _i in enumerate(range(_start, 26, 2)):
        _plan.append((0 if _i < 13 else 1, (_i % 13) * EMBED, _j * EMBED))
    _PLANS.append(tuple(_plan))
_PLANS = tuple(_PLANS)


def _make_kernel():
    info = plsc.get_sparse_core_info()
    nc, ns = info.num_cores, info.num_subcores
    nw = nc * ns
    rpw = ROWS // nw
    nchunks = rpw // CHUNK

    mesh = plsc.VectorSubcoreMesh(core_axis_name="c", subcore_axis_name="s")

    @functools.partial(
        pl.kernel,
        mesh=mesh,
        out_type=[
            jax.ShapeDtypeStruct((ROWS, WIDTH), jnp.float32),
            jax.ShapeDtypeStruct((ROWS, WIDTH), jnp.float32),
        ],
        scratch_types=[
            pltpu.VMEM((CHUNK, WIDTH), jnp.float32),
            pltpu.VMEM((CHUNK, WIDTH), jnp.float32),
            pltpu.SemaphoreType.DMA,
            pltpu.SemaphoreType.DMA,
        ],
    )
    def regroup(kt0, kt1, out_even, out_odd, buf_a, buf_b, gsem, ssem):
        wid = lax.axis_index("s") * nc + lax.axis_index("c")
        r0 = wid * rpw
        srcs = (kt0, kt1)
        outs = (out_even, out_odd)
        bufs = (buf_a, buf_b)
        items = [(o, c) for o in range(2) for c in range(nchunks)]
        nit = len(items)

        def fire_gathers(item_idx, buf):
            o, c = items[item_idx]
            rows = r0 + c * CHUNK
            handles = []
            for src, sc, dc in _PLANS[o]:
                handles.append(
                    pltpu.async_copy(
                        srcs[src].at[pl.ds(rows, CHUNK), pl.ds(sc, EMBED)],
                        buf.at[:, pl.ds(dc, EMBED)],
                        gsem,
                    )
                )
            return handles

        scatters = [None] * nit
        cur_g = fire_gathers(0, bufs[0])
        for it in range(nit):
            for h in cur_g:
                h.wait()
            o, c = items[it]
            rows = r0 + c * CHUNK
            scatters[it] = pltpu.async_copy(
                bufs[it % 2], outs[o].at[pl.ds(rows, CHUNK), :], ssem
            )
            if it + 1 < nit:
                if it >= 1:
                    scatters[it - 1].wait()
                cur_g = fire_gathers(it + 1, bufs[(it + 1) % 2])
        scatters[nit - 2].wait()
        scatters[nit - 1].wait()

    return regroup


_REGROUP = None


def kernel(kt0_values, kt1_values):
    global _REGROUP
    if _REGROUP is None:
        _REGROUP = _make_kernel()
    return tuple(_REGROUP(kt0_values, kt1_values))


# speedup vs baseline: 26.8675x; 1.0060x over previous
import functools

import jax
import jax.numpy as jnp
from jax import lax
from jax.experimental import pallas as pl
from jax.experimental.pallas import tpu as pltpu
from jax.experimental.pallas import tpu_sc as plsc

EMBED = 128
ROWS = 4096
WIDTH = 13 * EMBED
CHUNK = 16
NBUF = 3
DEPTH = 2

_PLANS = []
for _start in (0, 1):
    _plan = []
    for _j, _i in enumerate(range(_start, 26, 2)):
        _plan.append((0 if _i < 13 else 1, (_i % 13) * EMBED, _j * EMBED))
    _PLANS.append(tuple(_plan))
_PLANS = tuple(_PLANS)


def _make_kernel():
    info = plsc.get_sparse_core_info()
    nc, ns = info.num_cores, info.num_subcores
    nw = nc * ns
    rpw = ROWS // nw
    nchunks = rpw // CHUNK

    mesh = plsc.VectorSubcoreMesh(core_axis_name="c", subcore_axis_name="s")

    @functools.partial(
        pl.kernel,
        mesh=mesh,
        out_type=[
            jax.ShapeDtypeStruct((ROWS, WIDTH), jnp.float32),
            jax.ShapeDtypeStruct((ROWS, WIDTH), jnp.float32),
        ],
        scratch_types=(
            [pltpu.VMEM((CHUNK, WIDTH), jnp.float32) for _ in range(NBUF)]
            + [pltpu.SemaphoreType.DMA for _ in range(DEPTH + NBUF)]
        ),
    )
    def regroup(kt0, kt1, out_even, out_odd, *scratch):
        bufs = scratch[:NBUF]
        gsems = scratch[NBUF : NBUF + DEPTH]
        ssems = scratch[NBUF + DEPTH :]
        wid = lax.axis_index("s") * nc + lax.axis_index("c")
        r0 = wid * rpw
        srcs = (kt0, kt1)
        outs = (out_even, out_odd)
        items = [(o, c) for o in range(2) for c in range(nchunks)]
        nit = len(items)

        def fire_gathers(item_idx):
            o, c = items[item_idx]
            rows = r0 + c * CHUNK
            buf = bufs[item_idx % NBUF]
            sem = gsems[item_idx % DEPTH]
            handles = []
            for src, sc, dc in _PLANS[o]:
                handles.append(
                    pltpu.async_copy(
                        srcs[src].at[pl.ds(rows, CHUNK), pl.ds(sc, EMBED)],
                        buf.at[:, pl.ds(dc, EMBED)],
                        sem,
                    )
                )
            return handles

        gh = {}
        for k in range(DEPTH):
            gh[k] = fire_gathers(k)
        sh = [None] * nit
        for it in range(nit):
            for h in gh.pop(it):
                h.wait()
            o, c = items[it]
            rows = r0 + c * CHUNK
            sh[it] = pltpu.async_copy(
                bufs[it % NBUF], outs[o].at[pl.ds(rows, CHUNK), :], ssems[it % NBUF]
            )
            nx = it + DEPTH
            if nx < nit:
                if nx - NBUF >= 0:
                    sh[nx - NBUF].wait()
                gh[nx] = fire_gathers(nx)
        for j in range(nit - NBUF, nit):
            sh[j].wait()

    return regroup


_REGROUP = None


def kernel(kt0_values, kt1_values):
    global _REGROUP
    if _REGROUP is None:
        _REGROUP = _make_kernel()
    return tuple(_REGROUP(kt0_values, kt1_values))
